# baseline (speedup 1.0000x reference)
import functools

import jax
import jax.numpy as jnp
from jax import lax
from jax.experimental import pallas as pl
from jax.experimental.pallas import tpu as pltpu
from jax.experimental.pallas import tpu_sc as plsc

N = 10000
E = 320000
IN_DIM = 128
HID = 256
OUT = 128
NB = 64
BS = 100
NC = 2
TEMP = 0.5

NCORES = 2
NSUB = 16
EW = E // 128
ROW_BLK = 400
BLK = 2000

_PREC = lax.Precision.DEFAULT


def _mesh():
    return plsc.VectorSubcoreMesh(core_axis_name="c", subcore_axis_name="s")


def _gelu(x):
    return 0.5 * x * (1.0 + lax.erf(x * (2.0 ** -0.5)))


def _layernorm(x, g, b):
    mu = x.mean(-1, keepdims=True)
    var = ((x - mu) ** 2).mean(-1, keepdims=True)
    return (x - mu) / jnp.sqrt(var + 1e-5) * g + b


def _dot(a, b):
    return lax.dot_general(a, b, (((1,), (0,)), ((), ())),
                           preferred_element_type=jnp.float32,
                           precision=_PREC)


def _row_part(s, n_out, fn):
    if n_out // NSUB >= 8:
        ch = (n_out // NSUB) // 8 * 8
        fn(pl.multiple_of(s * ch, 8), ch)
        tail = n_out - ch * NSUB
        if tail:
            @pl.when(s == 0)
            def _():
                fn(ch * NSUB, tail)
    else:
        nse = n_out // 8

        @pl.when(s < nse)
        def _():
            fn(pl.multiple_of(s * 8, 8), 8)


def _widx(w):
    return pl.ds(pl.multiple_of(w * 128, 8), 128)


SLAB = 16


def _slab_engine(acc, cidx, rows0, rows1, sems, wbase, lim, gather_fn):
    bufs = (rows0, rows1)
    descs = [None, None]
    for j in range(SLAB):
        if gather_fn is not None:
            descs[j % 2] = pltpu.async_copy(
                gather_fn(cidx.at[2 * j]), bufs[j % 2], sems[j % 2])
        if j > 0:
            jj = j - 1
            if gather_fn is not None:
                descs[jj % 2].wait()

            @pl.when(wbase + jj < lim)
            def _(jj=jj):
                pltpu.sync_copy(bufs[jj % 2] if gather_fn is not None
                                else rows0,
                                acc.at[cidx.at[2 * jj + 1]], add=True)
    jj = SLAB - 1
    if gather_fn is not None:
        descs[jj % 2].wait()

    @pl.when(wbase + jj < lim)
    def _():
        pltpu.sync_copy(bufs[jj % 2] if gather_fn is not None else rows0,
                        acc.at[cidx.at[2 * jj + 1]], add=True)


def _rup(x, m):
    return -(-x // m) * m


def _sc_hist(idx2w, zeros, ones, ew):
    pw = idx2w.shape[0] // 2
    wpt = _rup(_rup(ew, NSUB) // NSUB, SLAB)

    @functools.partial(
        pl.kernel,
        out_type=jax.ShapeDtypeStruct((NCORES, N, 128), jnp.float32),
        mesh=_mesh(),
        scratch_types=[
            pltpu.VMEM((SLAB, 128), jnp.int32),
            pltpu.VMEM((128, 128), jnp.float32),
            pltpu.VMEM_SHARED((N, 128), jnp.float32),
        ],
    )
    def k(idx_h, z_h, ones_h, out_h, didx, ones_v, hist):
        c = lax.axis_index("c")
        s = lax.axis_index("s")
        pltpu.sync_copy(ones_h, ones_v)
        _row_part(s, N, lambda st, sz: pltpu.sync_copy(
            z_h.at[pl.ds(st, sz)], hist.at[pl.ds(st, sz)]))
        plsc.subcore_barrier()
        base = s * wpt
        crow = c * pw

        @pl.loop(0, wpt // SLAB)
        def _(kk):
            wbase = base + kk * SLAB

            @pl.when(wbase < ew)
            def _():
                pltpu.sync_copy(
                    idx_h.at[pl.ds(pl.multiple_of(crow + wbase, 8), SLAB)],
                    didx)
                for j in range(SLAB):
                    @pl.when(wbase + j < ew)
                    def _(j=j):
                        pltpu.sync_copy(ones_v, hist.at[didx.at[j]],
                                        add=True)

        plsc.subcore_barrier()
        _row_part(s, N, lambda st, sz: pltpu.sync_copy(
            hist.at[pl.ds(st, sz)], out_h.at[c, pl.ds(st, sz)]))

    return k(idx2w, zeros, ones)


def _sc_segsum_split(y, cidx2, zeros, ew):
    n_out = zeros.shape[0]
    wpc = _rup(_rup(ew, NCORES) // NCORES, SLAB)
    wpt = _rup(_rup(wpc, NSUB) // NSUB, SLAB)

    @functools.partial(
        pl.kernel,
        out_type=jax.ShapeDtypeStruct((NCORES, n_out, 128), jnp.float32),
        mesh=_mesh(),
        scratch_types=[
            pltpu.VMEM((2 * SLAB, 128), jnp.int32),
            pltpu.VMEM((128, 128), jnp.float32),
            pltpu.VMEM((128, 128), jnp.float32),
            pltpu.SemaphoreType.DMA,
            pltpu.SemaphoreType.DMA,
            pltpu.VMEM_SHARED((n_out, 128), jnp.float32),
        ],
    )
    def k(y_h, cidx_h, z_h, out_h, cidx, rows0, rows1, sem0, sem1, acc):
        c = lax.axis_index("c")
        s = lax.axis_index("s")
        _row_part(s, n_out, lambda st, sz: pltpu.sync_copy(
            z_h.at[pl.ds(st, sz)], acc.at[pl.ds(st, sz)]))
        plsc.subcore_barrier()
        lim = jnp.minimum((c + 1) * wpc, ew)
        base = c * wpc + s * wpt

        @pl.loop(0, wpt // SLAB)
        def _(kk):
            wbase = base + kk * SLAB

            @pl.when(wbase < lim)
            def _():
                pltpu.sync_copy(
                    cidx_h.at[pl.ds(pl.multiple_of(2 * wbase, 8),
                                    2 * SLAB)], cidx)
                _slab_engine(acc, cidx, rows0, rows1, (sem0, sem1), wbase,
                             lim, lambda iv: y_h.at[iv])

        plsc.subcore_barrier()
        _row_part(s, n_out, lambda st, sz: pltpu.sync_copy(
            acc.at[pl.ds(st, sz)], out_h.at[c, pl.ds(st, sz)]))

    return k(y, cidx2, zeros)


def _sc_segsum_feat(ycat, cidx4, zeros, ew):
    pw = cidx4.shape[0] // 4
    n_out = zeros.shape[0]
    wpt = _rup(_rup(ew, NSUB) // NSUB, SLAB)

    @functools.partial(
        pl.kernel,
        out_type=jax.ShapeDtypeStruct((NCORES, n_out, 128), jnp.float32),
        mesh=_mesh(),
        scratch_types=[
            pltpu.VMEM((2 * SLAB, 128), jnp.int32),
            pltpu.VMEM((128, 128), jnp.float32),
            pltpu.VMEM((128, 128), jnp.float32),
            pltpu.SemaphoreType.DMA,
            pltpu.SemaphoreType.DMA,
            pltpu.VMEM_SHARED((n_out, 128), jnp.float32),
        ],
    )
    def k(ycat_h, cidx_h, z_h, out_h, cidx, rows0, rows1, sem0, sem1, acc):
        c = lax.axis_index("c")
        s = lax.axis_index("s")
        _row_part(s, n_out, lambda st, sz: pltpu.sync_copy(
            z_h.at[pl.ds(st, sz)], acc.at[pl.ds(st, sz)]))
        plsc.subcore_barrier()
        base = s * wpt
        crow = c * (2 * pw)

        @pl.loop(0, wpt // SLAB)
        def _(kk):
            wbase = base + kk * SLAB

            @pl.when(wbase < ew)
            def _():
                pltpu.sync_copy(
                    cidx_h.at[pl.ds(pl.multiple_of(crow + 2 * wbase, 8),
                                    2 * SLAB)], cidx)
                _slab_engine(acc, cidx, rows0, rows1, (sem0, sem1), wbase,
                             ew, lambda iv: ycat_h.at[iv])

        plsc.subcore_barrier()
        _row_part(s, n_out, lambda st, sz: pltpu.sync_copy(
            acc.at[pl.ds(st, sz)], out_h.at[c, pl.ds(st, sz)]))

    return k(ycat, cidx4, zeros)


def _ns_from(hist_blk):
    return lax.rsqrt(jnp.maximum(hist_blk[:, :1], 1.0))


def _prep_body(feat_r, hs_r, y_r):
    y_r[...] = feat_r[...] * _ns_from(hs_r[...])


def _prep(feat, hs):
    return pl.pallas_call(
        _prep_body,
        grid=(N // BLK,),
        in_specs=[
            pl.BlockSpec((BLK, 128), lambda i: (i, 0)),
            pl.BlockSpec((BLK, 128), lambda i: (i, 0)),
        ],
        out_specs=pl.BlockSpec((BLK, 128), lambda i: (i, 0)),
        out_shape=jax.ShapeDtypeStruct((N, 128), jnp.float32),
    )(feat, hs)


def _post1_body(a0_r, a1_r, hd_r, hs_r, W1_r, b1_r, y2_r):
    nd = _ns_from(hd_r[...])
    ns = _ns_from(hs_r[...])
    agg = (a0_r[...] + a1_r[...]) * nd
    h = _gelu(_dot(agg, W1_r[...]) + b1_r[...])
    y = h * ns
    y2_r[0] = y[:, :128]
    y2_r[1] = y[:, 128:]


def _post1(a0, a1, hd, hs, W1, b1):
    return pl.pallas_call(
        _post1_body,
        grid=(N // BLK,),
        in_specs=[
            pl.BlockSpec((BLK, 128), lambda i: (i, 0)),
            pl.BlockSpec((BLK, 128), lambda i: (i, 0)),
            pl.BlockSpec((BLK, 128), lambda i: (i, 0)),
            pl.BlockSpec((BLK, 128), lambda i: (i, 0)),
            pl.BlockSpec((128, HID), lambda i: (0, 0)),
            pl.BlockSpec((1, HID), lambda i: (0, 0)),
        ],
        out_specs=pl.BlockSpec((2, BLK, 128), lambda i: (0, i, 0)),
        out_shape=jax.ShapeDtypeStruct((2, N, 128), jnp.float32),
    )(a0, a1, hd, hs, W1, b1.reshape(1, HID))


def _post2_body(a0_r, a1_r, hd_r, hs_r, W2_r, b2_r, W3_r, t_r):
    nd = _ns_from(hd_r[...])
    ns = _ns_from(hs_r[...])
    W2 = W2_r[...]
    u = _gelu(_dot(a0_r[...] * nd, W2[:128, :]) +
              _dot(a1_r[...] * nd, W2[128:, :]) + b2_r[...])
    t_r[...] = _dot(u, W3_r[...]) * ns


def _post2(a0, a1, hd, hs, W2, b2, W3):
    return pl.pallas_call(
        _post2_body,
        grid=(N // BLK,),
        in_specs=[
            pl.BlockSpec((BLK, 128), lambda i: (i, 0)),
            pl.BlockSpec((BLK, 128), lambda i: (i, 0)),
            pl.BlockSpec((BLK, 128), lambda i: (i, 0)),
            pl.BlockSpec((BLK, 128), lambda i: (i, 0)),
            pl.BlockSpec((HID, HID), lambda i: (0, 0)),
            pl.BlockSpec((1, HID), lambda i: (0, 0)),
            pl.BlockSpec((HID, 128), lambda i: (0, 0)),
        ],
        out_specs=pl.BlockSpec((BLK, 128), lambda i: (i, 0)),
        out_shape=jax.ShapeDtypeStruct((N, 128), jnp.float32),
    )(a0, a1, hd, hs, W2, b2.reshape(1, HID), W3)


def _post3_body(a0_r, a1_r, hd_r, b3_r, lng_r, lnb_r, Wp1_r, bp1_r, Wp2_r,
                bp2_r, Wh1_r, bh1_r, wv_r, cb_r, h_r, z_r, sc_r):
    nd = _ns_from(hd_r[...])
    hh = _layernorm((a0_r[...] + a1_r[...]) * nd + b3_r[...],
                    lng_r[...], lnb_r[...])
    h_r[...] = hh
    zz = _dot(_gelu(_dot(hh, Wp1_r[...]) + bp1_r[...]), Wp2_r[...]) + bp2_r[...]
    nrm = jnp.sqrt(jnp.sum(zz * zz, axis=1, keepdims=True))
    z_r[...] = zz / jnp.maximum(nrm, 1e-12)
    A = _gelu(_dot(hh, Wh1_r[...]) + bh1_r[...])
    sc_r[...] = _dot(A, wv_r[...]) + cb_r[...]


def _post3(a0, a1, hd, b3, ln_g, ln_b, Wp1, bp1, Wp2, bp2, Wh1f, bh1f, wvp, cbv):
    row = pl.BlockSpec((BLK, 128), lambda i: (i, 0))
    c128 = pl.BlockSpec((1, 128), lambda i: (0, 0))
    return pl.pallas_call(
        _post3_body,
        grid=(N // BLK,),
        in_specs=[
            row, row,
            pl.BlockSpec((BLK, 128), lambda i: (i, 0)),
            c128, c128, c128,
            pl.BlockSpec((128, 128), lambda i: (0, 0)), c128,
            pl.BlockSpec((128, 128), lambda i: (0, 0)), c128,
            pl.BlockSpec((128, 512), lambda i: (0, 0)),
            pl.BlockSpec((1, 512), lambda i: (0, 0)),
            pl.BlockSpec((512, 128), lambda i: (0, 0)),
            c128,
        ],
        out_specs=[row, row, row],
        out_shape=[
            jax.ShapeDtypeStruct((N, 128), jnp.float32),
            jax.ShapeDtypeStruct((N, 128), jnp.float32),
            jax.ShapeDtypeStruct((N, 128), jnp.float32),
        ],
    )(a0, a1, hd, b3.reshape(1, 128), ln_g.reshape(1, 128),
      ln_b.reshape(1, 128), Wp1, bp1.reshape(1, 128), Wp2,
      bp2.reshape(1, 128), Wh1f, bh1f, wvp, cbv)


def _softmaxw_body(sc_r, h_r, w_r):
    s0 = sc_r[:, :1]
    m = jnp.max(s0)
    e = jnp.exp(s0 - m)
    w = e / jnp.sum(e)
    w_r[...] = w * h_r[...]


def _softmaxw(sc, h):
    return pl.pallas_call(
        _softmaxw_body,
        in_specs=[
            pl.BlockSpec((N, 128), lambda: (0, 0)),
            pl.BlockSpec((N, 128), lambda: (0, 0)),
        ],
        out_specs=pl.BlockSpec((N, 128), lambda: (0, 0)),
        out_shape=jax.ShapeDtypeStruct((N, 128), jnp.float32),
    )(sc, h)


def _contrast_kernel(z_blk_ref, z_all_ref, out_ref):
    i = pl.program_id(0)
    z_blk = z_blk_ref[...]
    s = lax.dot_general(
        z_blk, z_all_ref[...], (((1,), (1,)), ((), ())),
        preferred_element_type=jnp.float32,
        precision=_PREC,
    ) * (1.0 / TEMP)
    m = jnp.max(s, axis=1, keepdims=True)
    lse = jnp.log(jnp.sum(jnp.exp(s - m), axis=1, keepdims=True)) + m
    diag = jnp.sum(z_blk * z_blk, axis=1, keepdims=True) * (1.0 / TEMP)
    part = jnp.sum(lse - diag, axis=(0, 1), keepdims=True)

    @pl.when(i == 0)
    def _():
        out_ref[...] = jnp.zeros_like(out_ref)

    out_ref[...] += part


def _contrast(z):
    return pl.pallas_call(
        _contrast_kernel,
        grid=(N // ROW_BLK,),
        in_specs=[
            pl.BlockSpec((ROW_BLK, OUT), lambda i: (i, 0)),
            pl.BlockSpec((N, OUT), lambda i: (0, 0)),
        ],
        out_specs=pl.BlockSpec((1, 1), lambda i: (0, 0)),
        out_shape=jax.ShapeDtypeStruct((1, 1), jnp.float32),
    )(z, z)


def _final_body(b0_r, b1_r, Wc1_r, bc1_r, lg_r, lb_r, Wc2_r, bc2_r, oh_r,
                tot_r, logits_r, loss_r):
    bf = b0_r[...] + b1_r[...]
    x = _gelu(_layernorm(_dot(bf, Wc1_r[...]) + bc1_r[...], lg_r[...], lb_r[...]))
    logits = _dot(x, Wc2_r[...]) + bc2_r[...]
    logits_r[...] = logits
    m = jnp.max(logits, axis=1, keepdims=True)
    lse = jnp.log(jnp.sum(jnp.exp(logits - m), axis=1, keepdims=True)) + m
    lp = logits - lse
    nll = -jnp.sum(lp * oh_r[...], axis=1, keepdims=True)
    pmean = -jnp.mean(lp, axis=1, keepdims=True)
    cls = jnp.sum(0.9 * nll + 0.1 * pmean, axis=(0, 1), keepdims=True) / NB
    loss_r[...] = tot_r[...] * (0.6 / N) + cls * 0.4


def _final(b0, b1, Wc1, bc1, lnc_g, lnc_b, Wc2, bc2, oh, total):
    full = lambda s: pl.BlockSpec(s, lambda: tuple(0 for _ in s))
    return pl.pallas_call(
        _final_body,
        in_specs=[
            full((NB, 128)), full((NB, 128)),
            full((128, 128)), full((1, 128)), full((1, 128)), full((1, 128)),
            full((128, NC)), full((1, NC)), full((NB, NC)), full((1, 1)),
        ],
        out_specs=[full((NB, NC)), full((1, 1))],
        out_shape=[
            jax.ShapeDtypeStruct((NB, NC), jnp.float32),
            jax.ShapeDtypeStruct((1, 1), jnp.float32),
        ],
    )(b0, b1, Wc1, bc1.reshape(1, 128), lnc_g.reshape(1, 128),
      lnc_b.reshape(1, 128), Wc2, bc2.reshape(1, NC), oh, total)


def kernel(feat, edge_index, bag_indices, labels, W1, b1, W2, b2, W3, b3,
           ln_g, ln_b, Wh1, bh1, Wh2, bh2, Wp1, bp1, Wp2, bp2, Wc1, bc1,
           lnc_g, lnc_b, Wc2, bc2):
    PW = 2560
    srcw2 = jnp.pad(edge_index[0], (0, PW * 128 - E)).reshape(PW, 128)
    dstw2 = jnp.pad(edge_index[1], (0, PW * 128 - E)).reshape(PW, 128)
    zN = jnp.zeros((N, 128), jnp.float32)
    zB = jnp.zeros((NB, 128), jnp.float32)
    o128 = jnp.ones((128, 128), jnp.float32)

    cidx2 = jnp.stack([srcw2, dstw2], axis=1).reshape(2 * PW, 128)
    cidx4 = jnp.stack([jnp.stack([srcw2, dstw2], 1),
                       jnp.stack([srcw2 + N, dstw2], 1)]).reshape(4 * PW, 128)

    hist = _sc_hist(jnp.concatenate([srcw2, dstw2]), zN, o128, EW)
    hs, hd = hist[0], hist[1]

    y0 = _prep(feat, hs)
    agg1 = _sc_segsum_split(y0, cidx2, zN, EW)
    y2 = _post1(agg1[0], agg1[1], hd, hs, W1, b1)
    agg2 = _sc_segsum_feat(y2.reshape(2 * N, 128), cidx4, zN, EW)
    t = _post2(agg2[0], agg2[1], hd, hs, W2, b2, W3)
    agg3 = _sc_segsum_split(t, cidx2, zN, EW)

    Wh1f = Wh1.transpose(1, 0, 2).reshape(128, 512)
    bh1f = bh1.reshape(1, 512)
    wvp = jnp.pad(Wh2.reshape(512, 1) / 4.0, ((0, 0), (0, 127)))
    cbv = jnp.full((1, 128), jnp.mean(bh2), jnp.float32)
    h, z, sc = _post3(agg3[0], agg3[1], hd, b3, ln_g, ln_b, Wp1, bp1, Wp2,
                      bp2, Wh1f, bh1f, wvp, cbv)

    total = _contrast(z)
    weighted = _softmaxw(sc, h)

    PWB = 64
    bagw2 = jnp.pad(bag_indices.reshape(NB * BS),
                    (0, PWB * 128 - NB * BS)).reshape(PWB, 128)
    bagidw2 = jnp.pad(jnp.repeat(jnp.arange(NB, dtype=jnp.int32), BS),
                      (0, PWB * 128 - NB * BS)).reshape(PWB, 128)
    cidxb = jnp.stack([bagw2, bagidw2], axis=1).reshape(2 * PWB, 128)
    bagf = _sc_segsum_split(weighted, cidxb, zB, NB * BS // 128)

    oh = (labels[:, None] == jnp.arange(NC, dtype=labels.dtype)[None, :]
          ).astype(jnp.float32)
    logits, loss = _final(bagf[0], bagf[1], Wc1, bc1, lnc_g, lnc_b, Wc2, bc2,
                          oh, total)
    return logits, loss[0, 0]


# speedup vs baseline: 1.1196x; 1.1196x over previous
import functools

import jax
import jax.numpy as jnp
from jax import lax
from jax.experimental import pallas as pl
from jax.experimental.pallas import tpu as pltpu
from jax.experimental.pallas import tpu_sc as plsc

N = 10000
E = 320000
IN_DIM = 128
HID = 256
OUT = 128
NB = 64
BS = 100
NC = 2
TEMP = 0.5

NCORES = 2
NSUB = 16
EW = E // 128
ROW_BLK = 400
BLK = 2000

_PREC = lax.Precision.DEFAULT


def _mesh():
    return plsc.VectorSubcoreMesh(core_axis_name="c", subcore_axis_name="s")


def _gelu(x):
    return 0.5 * x * (1.0 + lax.erf(x * (2.0 ** -0.5)))


def _layernorm(x, g, b):
    mu = x.mean(-1, keepdims=True)
    var = ((x - mu) ** 2).mean(-1, keepdims=True)
    return (x - mu) / jnp.sqrt(var + 1e-5) * g + b


def _dot(a, b):
    return lax.dot_general(a, b, (((1,), (0,)), ((), ())),
                           preferred_element_type=jnp.float32,
                           precision=_PREC)


def _row_part(s, n_out, fn):
    if n_out // NSUB >= 8:
        ch = (n_out // NSUB) // 8 * 8
        fn(pl.multiple_of(s * ch, 8), ch)
        tail = n_out - ch * NSUB
        if tail:
            @pl.when(s == 0)
            def _():
                fn(ch * NSUB, tail)
    else:
        nse = n_out // 8

        @pl.when(s < nse)
        def _():
            fn(pl.multiple_of(s * 8, 8), 8)


def _widx(w):
    return pl.ds(pl.multiple_of(w * 128, 8), 128)


SLAB = 8


def _slab_engine(acc, cidx, rows0, rows1, sems, wbase, lim, gather_fn):
    bufs = (rows0, rows1)
    descs = [None, None]
    for j in range(SLAB):
        if gather_fn is not None:
            descs[j % 2] = pltpu.async_copy(
                gather_fn(cidx.at[2 * j]), bufs[j % 2], sems[j % 2])
        if j > 0:
            jj = j - 1
            if gather_fn is not None:
                descs[jj % 2].wait()

            @pl.when(wbase + jj < lim)
            def _(jj=jj):
                pltpu.sync_copy(bufs[jj % 2] if gather_fn is not None
                                else rows0,
                                acc.at[cidx.at[2 * jj + 1]], add=True)
    jj = SLAB - 1
    if gather_fn is not None:
        descs[jj % 2].wait()

    @pl.when(wbase + jj < lim)
    def _():
        pltpu.sync_copy(bufs[jj % 2] if gather_fn is not None else rows0,
                        acc.at[cidx.at[2 * jj + 1]], add=True)


def _rup(x, m):
    return -(-x // m) * m


def _sc_hist(idx2w, zeros, ones, ew):
    pw = idx2w.shape[0] // 2
    wpt = _rup(_rup(ew, NSUB) // NSUB, SLAB)

    @functools.partial(
        pl.kernel,
        out_type=jax.ShapeDtypeStruct((NCORES, N, 128), jnp.float32),
        mesh=_mesh(),
        scratch_types=[
            pltpu.VMEM((SLAB, 128), jnp.int32),
            pltpu.VMEM((128, 128), jnp.float32),
            pltpu.VMEM_SHARED((N, 128), jnp.float32),
        ],
    )
    def k(idx_h, z_h, ones_h, out_h, didx, ones_v, hist):
        c = lax.axis_index("c")
        s = lax.axis_index("s")
        pltpu.sync_copy(ones_h, ones_v)
        _row_part(s, N, lambda st, sz: pltpu.sync_copy(
            z_h.at[pl.ds(st, sz)], hist.at[pl.ds(st, sz)]))
        plsc.subcore_barrier()
        base = s * wpt
        crow = c * pw

        @pl.loop(0, wpt // SLAB)
        def _(kk):
            wbase = base + kk * SLAB

            @pl.when(wbase < ew)
            def _():
                pltpu.sync_copy(
                    idx_h.at[pl.ds(pl.multiple_of(crow + wbase, 8), SLAB)],
                    didx)
                for j in range(SLAB):
                    @pl.when(wbase + j < ew)
                    def _(j=j):
                        pltpu.sync_copy(ones_v, hist.at[didx.at[j]],
                                        add=True)

        plsc.subcore_barrier()
        _row_part(s, N, lambda st, sz: pltpu.sync_copy(
            hist.at[pl.ds(st, sz)], out_h.at[c, pl.ds(st, sz)]))

    return k(idx2w, zeros, ones)


def _sc_segsum_split(y, cidx2, zeros, ew):
    n_out = zeros.shape[0]
    wpc = _rup(_rup(ew, NCORES) // NCORES, SLAB)
    wpt = _rup(_rup(wpc, NSUB) // NSUB, SLAB)

    @functools.partial(
        pl.kernel,
        out_type=jax.ShapeDtypeStruct((NCORES, n_out, 128), jnp.float32),
        mesh=_mesh(),
        scratch_types=[
            pltpu.VMEM((2 * SLAB, 128), jnp.int32),
            pltpu.VMEM((128, 128), jnp.float32),
            pltpu.VMEM((128, 128), jnp.float32),
            pltpu.SemaphoreType.DMA,
            pltpu.SemaphoreType.DMA,
            pltpu.VMEM_SHARED((n_out, 128), jnp.float32),
        ],
    )
    def k(y_h, cidx_h, z_h, out_h, cidx, rows0, rows1, sem0, sem1, acc):
        c = lax.axis_index("c")
        s = lax.axis_index("s")
        _row_part(s, n_out, lambda st, sz: pltpu.sync_copy(
            z_h.at[pl.ds(st, sz)], acc.at[pl.ds(st, sz)]))
        plsc.subcore_barrier()
        lim = jnp.minimum((c + 1) * wpc, ew)
        base = c * wpc + s * wpt

        @pl.loop(0, wpt // SLAB)
        def _(kk):
            wbase = base + kk * SLAB

            @pl.when(wbase < lim)
            def _():
                pltpu.sync_copy(
                    cidx_h.at[pl.ds(pl.multiple_of(2 * wbase, 8),
                                    2 * SLAB)], cidx)
                _slab_engine(acc, cidx, rows0, rows1, (sem0, sem1), wbase,
                             lim, lambda iv: y_h.at[iv])

        plsc.subcore_barrier()
        _row_part(s, n_out, lambda st, sz: pltpu.sync_copy(
            acc.at[pl.ds(st, sz)], out_h.at[c, pl.ds(st, sz)]))

    return k(y, cidx2, zeros)


def _sc_segsum_feat(ycat, cidx4, zeros, ew):
    pw = cidx4.shape[0] // 4
    n_out = zeros.shape[0]
    wpt = _rup(_rup(ew, NSUB) // NSUB, SLAB)

    @functools.partial(
        pl.kernel,
        out_type=jax.ShapeDtypeStruct((NCORES, n_out, 128), jnp.float32),
        mesh=_mesh(),
        scratch_types=[
            pltpu.VMEM((2 * SLAB, 128), jnp.int32),
            pltpu.VMEM((128, 128), jnp.float32),
            pltpu.VMEM((128, 128), jnp.float32),
            pltpu.SemaphoreType.DMA,
            pltpu.SemaphoreType.DMA,
            pltpu.VMEM_SHARED((n_out, 128), jnp.float32),
        ],
    )
    def k(ycat_h, cidx_h, z_h, out_h, cidx, rows0, rows1, sem0, sem1, acc):
        c = lax.axis_index("c")
        s = lax.axis_index("s")
        _row_part(s, n_out, lambda st, sz: pltpu.sync_copy(
            z_h.at[pl.ds(st, sz)], acc.at[pl.ds(st, sz)]))
        plsc.subcore_barrier()
        base = s * wpt
        crow = c * (2 * pw)

        @pl.loop(0, wpt // SLAB)
        def _(kk):
            wbase = base + kk * SLAB

            @pl.when(wbase < ew)
            def _():
                pltpu.sync_copy(
                    cidx_h.at[pl.ds(pl.multiple_of(crow + 2 * wbase, 8),
                                    2 * SLAB)], cidx)
                _slab_engine(acc, cidx, rows0, rows1, (sem0, sem1), wbase,
                             ew, lambda iv: ycat_h.at[iv])

        plsc.subcore_barrier()
        _row_part(s, n_out, lambda st, sz: pltpu.sync_copy(
            acc.at[pl.ds(st, sz)], out_h.at[c, pl.ds(st, sz)]))

    return k(ycat, cidx4, zeros)


def _ns_from(hist_blk):
    return lax.rsqrt(jnp.maximum(hist_blk[:, :1], 1.0))


def _prep_body(feat_r, hs_r, y_r):
    y_r[...] = feat_r[...] * _ns_from(hs_r[...])


def _prep(feat, hs):
    return pl.pallas_call(
        _prep_body,
        grid=(N // BLK,),
        in_specs=[
            pl.BlockSpec((BLK, 128), lambda i: (i, 0)),
            pl.BlockSpec((BLK, 128), lambda i: (i, 0)),
        ],
        out_specs=pl.BlockSpec((BLK, 128), lambda i: (i, 0)),
        out_shape=jax.ShapeDtypeStruct((N, 128), jnp.float32),
    )(feat, hs)


def _post1_body(a0_r, a1_r, hd_r, hs_r, W1_r, b1_r, y2_r):
    nd = _ns_from(hd_r[...])
    ns = _ns_from(hs_r[...])
    agg = (a0_r[...] + a1_r[...]) * nd
    h = _gelu(_dot(agg, W1_r[...]) + b1_r[...])
    y = h * ns
    y2_r[0] = y[:, :128]
    y2_r[1] = y[:, 128:]


def _post1(a0, a1, hd, hs, W1, b1):
    return pl.pallas_call(
        _post1_body,
        grid=(N // BLK,),
        in_specs=[
            pl.BlockSpec((BLK, 128), lambda i: (i, 0)),
            pl.BlockSpec((BLK, 128), lambda i: (i, 0)),
            pl.BlockSpec((BLK, 128), lambda i: (i, 0)),
            pl.BlockSpec((BLK, 128), lambda i: (i, 0)),
            pl.BlockSpec((128, HID), lambda i: (0, 0)),
            pl.BlockSpec((1, HID), lambda i: (0, 0)),
        ],
        out_specs=pl.BlockSpec((2, BLK, 128), lambda i: (0, i, 0)),
        out_shape=jax.ShapeDtypeStruct((2, N, 128), jnp.float32),
    )(a0, a1, hd, hs, W1, b1.reshape(1, HID))


def _post2_body(a0_r, a1_r, hd_r, hs_r, W2_r, b2_r, W3_r, t_r):
    nd = _ns_from(hd_r[...])
    ns = _ns_from(hs_r[...])
    W2 = W2_r[...]
    u = _gelu(_dot(a0_r[...] * nd, W2[:128, :]) +
              _dot(a1_r[...] * nd, W2[128:, :]) + b2_r[...])
    t_r[...] = _dot(u, W3_r[...]) * ns


def _post2(a0, a1, hd, hs, W2, b2, W3):
    return pl.pallas_call(
        _post2_body,
        grid=(N // BLK,),
        in_specs=[
            pl.BlockSpec((BLK, 128), lambda i: (i, 0)),
            pl.BlockSpec((BLK, 128), lambda i: (i, 0)),
            pl.BlockSpec((BLK, 128), lambda i: (i, 0)),
            pl.BlockSpec((BLK, 128), lambda i: (i, 0)),
            pl.BlockSpec((HID, HID), lambda i: (0, 0)),
            pl.BlockSpec((1, HID), lambda i: (0, 0)),
            pl.BlockSpec((HID, 128), lambda i: (0, 0)),
        ],
        out_specs=pl.BlockSpec((BLK, 128), lambda i: (i, 0)),
        out_shape=jax.ShapeDtypeStruct((N, 128), jnp.float32),
    )(a0, a1, hd, hs, W2, b2.reshape(1, HID), W3)


def _post3_body(a0_r, a1_r, hd_r, b3_r, lng_r, lnb_r, Wp1_r, bp1_r, Wp2_r,
                bp2_r, Wh1_r, bh1_r, wv_r, cb_r, h_r, z_r, sc_r):
    nd = _ns_from(hd_r[...])
    hh = _layernorm((a0_r[...] + a1_r[...]) * nd + b3_r[...],
                    lng_r[...], lnb_r[...])
    h_r[...] = hh
    zz = _dot(_gelu(_dot(hh, Wp1_r[...]) + bp1_r[...]), Wp2_r[...]) + bp2_r[...]
    nrm = jnp.sqrt(jnp.sum(zz * zz, axis=1, keepdims=True))
    z_r[...] = zz / jnp.maximum(nrm, 1e-12)
    A = _gelu(_dot(hh, Wh1_r[...]) + bh1_r[...])
    sc_r[...] = _dot(A, wv_r[...]) + cb_r[...]


def _post3(a0, a1, hd, b3, ln_g, ln_b, Wp1, bp1, Wp2, bp2, Wh1f, bh1f, wvp, cbv):
    row = pl.BlockSpec((BLK, 128), lambda i: (i, 0))
    c128 = pl.BlockSpec((1, 128), lambda i: (0, 0))
    return pl.pallas_call(
        _post3_body,
        grid=(N // BLK,),
        in_specs=[
            row, row,
            pl.BlockSpec((BLK, 128), lambda i: (i, 0)),
            c128, c128, c128,
            pl.BlockSpec((128, 128), lambda i: (0, 0)), c128,
            pl.BlockSpec((128, 128), lambda i: (0, 0)), c128,
            pl.BlockSpec((128, 512), lambda i: (0, 0)),
            pl.BlockSpec((1, 512), lambda i: (0, 0)),
            pl.BlockSpec((512, 128), lambda i: (0, 0)),
            c128,
        ],
        out_specs=[row, row, row],
        out_shape=[
            jax.ShapeDtypeStruct((N, 128), jnp.float32),
            jax.ShapeDtypeStruct((N, 128), jnp.float32),
            jax.ShapeDtypeStruct((N, 128), jnp.float32),
        ],
    )(a0, a1, hd, b3.reshape(1, 128), ln_g.reshape(1, 128),
      ln_b.reshape(1, 128), Wp1, bp1.reshape(1, 128), Wp2,
      bp2.reshape(1, 128), Wh1f, bh1f, wvp, cbv)


def _softmaxw_body(sc_r, h_r, w_r):
    s0 = sc_r[:, :1]
    m = jnp.max(s0)
    e = jnp.exp(s0 - m)
    w = e / jnp.sum(e)
    w_r[...] = w * h_r[...]


def _softmaxw(sc, h):
    return pl.pallas_call(
        _softmaxw_body,
        in_specs=[
            pl.BlockSpec((N, 128), lambda: (0, 0)),
            pl.BlockSpec((N, 128), lambda: (0, 0)),
        ],
        out_specs=pl.BlockSpec((N, 128), lambda: (0, 0)),
        out_shape=jax.ShapeDtypeStruct((N, 128), jnp.float32),
    )(sc, h)


def _contrast_kernel(z_blk_ref, z_all_ref, out_ref):
    i = pl.program_id(0)
    z_blk = z_blk_ref[...]
    s = lax.dot_general(
        z_blk, z_all_ref[...], (((1,), (1,)), ((), ())),
        preferred_element_type=jnp.float32,
        precision=_PREC,
    ) * (1.0 / TEMP)
    m = jnp.max(s, axis=1, keepdims=True)
    lse = jnp.log(jnp.sum(jnp.exp(s - m), axis=1, keepdims=True)) + m
    diag = jnp.sum(z_blk * z_blk, axis=1, keepdims=True) * (1.0 / TEMP)
    part = jnp.sum(lse - diag, axis=(0, 1), keepdims=True)

    @pl.when(i == 0)
    def _():
        out_ref[...] = jnp.zeros_like(out_ref)

    out_ref[...] += part


def _contrast(z):
    return pl.pallas_call(
        _contrast_kernel,
        grid=(N // ROW_BLK,),
        in_specs=[
            pl.BlockSpec((ROW_BLK, OUT), lambda i: (i, 0)),
            pl.BlockSpec((N, OUT), lambda i: (0, 0)),
        ],
        out_specs=pl.BlockSpec((1, 1), lambda i: (0, 0)),
        out_shape=jax.ShapeDtypeStruct((1, 1), jnp.float32),
    )(z, z)


def _final_body(b0_r, b1_r, Wc1_r, bc1_r, lg_r, lb_r, Wc2_r, bc2_r, oh_r,
                tot_r, logits_r, loss_r):
    bf = b0_r[...] + b1_r[...]
    x = _gelu(_layernorm(_dot(bf, Wc1_r[...]) + bc1_r[...], lg_r[...], lb_r[...]))
    logits = _dot(x, Wc2_r[...]) + bc2_r[...]
    logits_r[...] = logits
    m = jnp.max(logits, axis=1, keepdims=True)
    lse = jnp.log(jnp.sum(jnp.exp(logits - m), axis=1, keepdims=True)) + m
    lp = logits - lse
    nll = -jnp.sum(lp * oh_r[...], axis=1, keepdims=True)
    pmean = -jnp.mean(lp, axis=1, keepdims=True)
    cls = jnp.sum(0.9 * nll + 0.1 * pmean, axis=(0, 1), keepdims=True) / NB
    loss_r[...] = tot_r[...] * (0.6 / N) + cls * 0.4


def _final(b0, b1, Wc1, bc1, lnc_g, lnc_b, Wc2, bc2, oh, total):
    full = lambda s: pl.BlockSpec(s, lambda: tuple(0 for _ in s))
    return pl.pallas_call(
        _final_body,
        in_specs=[
            full((NB, 128)), full((NB, 128)),
            full((128, 128)), full((1, 128)), full((1, 128)), full((1, 128)),
            full((128, NC)), full((1, NC)), full((NB, NC)), full((1, 1)),
        ],
        out_specs=[full((NB, NC)), full((1, 1))],
        out_shape=[
            jax.ShapeDtypeStruct((NB, NC), jnp.float32),
            jax.ShapeDtypeStruct((1, 1), jnp.float32),
        ],
    )(b0, b1, Wc1, bc1.reshape(1, 128), lnc_g.reshape(1, 128),
      lnc_b.reshape(1, 128), Wc2, bc2.reshape(1, NC), oh, total)


def kernel(feat, edge_index, bag_indices, labels, W1, b1, W2, b2, W3, b3,
           ln_g, ln_b, Wh1, bh1, Wh2, bh2, Wp1, bp1, Wp2, bp2, Wc1, bc1,
           lnc_g, lnc_b, Wc2, bc2):
    PW = 2560
    srcw2 = jnp.pad(edge_index[0], (0, PW * 128 - E)).reshape(PW, 128)
    dstw2 = jnp.pad(edge_index[1], (0, PW * 128 - E)).reshape(PW, 128)
    zN = jnp.zeros((N, 128), jnp.float32)
    zB = jnp.zeros((NB, 128), jnp.float32)
    o128 = jnp.ones((128, 128), jnp.float32)

    cidx2 = jnp.stack([srcw2, dstw2], axis=1).reshape(2 * PW, 128)
    cidx4 = jnp.stack([jnp.stack([srcw2, dstw2], 1),
                       jnp.stack([srcw2 + N, dstw2], 1)]).reshape(4 * PW, 128)

    hist = _sc_hist(jnp.concatenate([srcw2, dstw2]), zN, o128, EW)
    hs, hd = hist[0], hist[1]

    y0 = _prep(feat, hs)
    agg1 = _sc_segsum_split(y0, cidx2, zN, EW)
    y2 = _post1(agg1[0], agg1[1], hd, hs, W1, b1)
    agg2 = _sc_segsum_feat(y2.reshape(2 * N, 128), cidx4, zN, EW)
    t = _post2(agg2[0], agg2[1], hd, hs, W2, b2, W3)
    agg3 = _sc_segsum_split(t, cidx2, zN, EW)

    Wh1f = Wh1.transpose(1, 0, 2).reshape(128, 512)
    bh1f = bh1.reshape(1, 512)
    wvp = jnp.pad(Wh2.reshape(512, 1) / 4.0, ((0, 0), (0, 127)))
    cbv = jnp.full((1, 128), jnp.mean(bh2), jnp.float32)
    h, z, sc = _post3(agg3[0], agg3[1], hd, b3, ln_g, ln_b, Wp1, bp1, Wp2,
                      bp2, Wh1f, bh1f, wvp, cbv)

    total = _contrast(z)
    weighted = _softmaxw(sc, h)

    PWB = 64
    bagw2 = jnp.pad(bag_indices.reshape(NB * BS),
                    (0, PWB * 128 - NB * BS)).reshape(PWB, 128)
    bagidw2 = jnp.pad(jnp.repeat(jnp.arange(NB, dtype=jnp.int32), BS),
                      (0, PWB * 128 - NB * BS)).reshape(PWB, 128)
    cidxb = jnp.stack([bagw2, bagidw2], axis=1).reshape(2 * PWB, 128)
    bagf = _sc_segsum_split(weighted, cidxb, zB, NB * BS // 128)

    oh = (labels[:, None] == jnp.arange(NC, dtype=labels.dtype)[None, :]
          ).astype(jnp.float32)
    logits, loss = _final(bagf[0], bagf[1], Wc1, bc1, lnc_g, lnc_b, Wc2, bc2,
                          oh, total)
    return logits, loss[0, 0]
